# baseline (speedup 1.0000x reference)
import functools

import jax
import jax.numpy as jnp
import numpy as np
from jax import lax
from jax.experimental import pallas as pl
from jax.experimental.pallas import tpu as pltpu
from jax.experimental.pallas import tpu_sc as plsc

_N = 4096
_K = 30
_E = _N * _K
_T = 128
_ET = _T * _K
_GRID = _N // _T
_NUM_RBF = 16

_SC_COUNTS_ = np.array([1, 7, 4, 4, 2, 5, 5, 0, 6, 4, 4, 5, 4, 7, 3, 2, 3, 10, 8, 3])
_cum_ = np.concatenate([[0], np.cumsum(_SC_COUNTS_)])
_STARTS_ = 4 + _cum_[:-1]
_ENDS_ = 4 + _cum_[1:]
_crng_ = np.random.RandomState(7)
_LIT_POS_ = jnp.asarray(_crng_.randn(20, 14, 3).astype(np.float32))
_GROUP_IDX_ = jnp.asarray(_crng_.randint(0, 6, size=(20, 14)).astype(np.int32))

_f32 = jnp.float32


def _dot(a, b):
    return jnp.dot(a, b, preferred_element_type=_f32)


def _ln(x, g, b):
    mu = jnp.mean(x, -1, keepdims=True)
    var = jnp.mean((x - mu) ** 2, -1, keepdims=True)
    return (x - mu) / jnp.sqrt(var + 1e-5) * g + b


_NW = 32
_BPW = _E // _NW
_CH = 128
_NCH = _BPW // _CH


@functools.partial(jax.jit, static_argnums=(2,))
def _sc_gather(table, idx, d):
    mesh = plsc.VectorSubcoreMesh(core_axis_name="c", subcore_axis_name="s")

    @functools.partial(
        pl.kernel,
        mesh=mesh,
        out_type=jax.ShapeDtypeStruct((_E, d), _f32),
        scratch_types=[
            pltpu.VMEM((_BPW,), jnp.int32),
            pltpu.VMEM((_CH, d), _f32),
            pltpu.SemaphoreType.DMA,
        ],
    )
    def k(table_hbm, idx_hbm, out_hbm, idx_v, rows_v, sem):
        wid = lax.axis_index("s") * 2 + lax.axis_index("c")
        base = wid * _BPW
        pltpu.sync_copy(idx_hbm.at[pl.ds(base, _BPW)], idx_v)

        def body(j, carry):
            pltpu.async_copy(
                table_hbm.at[idx_v.at[pl.ds(j * _CH, _CH)]], rows_v, sem
            ).wait()
            pltpu.sync_copy(rows_v, out_hbm.at[pl.ds(base + j * _CH, _CH)])
            return carry

        lax.fori_loop(0, _NCH, body, 0)

    return k(table, idx)


def _embed_body(dists_ref, pe_ref, w1a_ref, w1b_ref, b1_ref, w2_ref, b2_ref,
                w3_ref, b3_ref, g_ref, bb_ref, out_ref):
    dists = dists_ref[...]
    col = lax.broadcasted_iota(jnp.int32, (25, 400), 1)
    row = lax.broadcasted_iota(jnp.int32, (25, 400), 0)
    S = (col // _NUM_RBF == row).astype(_f32)
    dexp = _dot(dists, S)
    cc = lax.broadcasted_iota(jnp.int32, (1, 400), 1)
    mu = 2.0 + (cc % _NUM_RBF).astype(_f32) * (20.0 / (_NUM_RBF - 1))
    rbf = jnp.exp(-(((dexp - mu) / 1.25) ** 2))
    h = jax.nn.relu(_dot(rbf, w1a_ref[...]) + _dot(pe_ref[...], w1b_ref[...])
                    + b1_ref[...])
    h = jax.nn.relu(_dot(h, w2_ref[...]) + b2_ref[...])
    h = _dot(h, w3_ref[...]) + b3_ref[...]
    out_ref[...] = _ln(h, g_ref[...], bb_ref[...])


def _edge_embed(dists, pe, pp):
    w1, b1 = pp['l1']['w'], pp['l1']['b']
    specs_w = lambda shp: pl.BlockSpec(shp, lambda i: (0, 0))
    return pl.pallas_call(
        _embed_body,
        grid=(_GRID,),
        in_specs=[
            pl.BlockSpec((_ET, 25), lambda i: (i, 0)),
            pl.BlockSpec((_ET, 16), lambda i: (i, 0)),
            specs_w((400, 256)), specs_w((16, 256)), specs_w((1, 256)),
            specs_w((256, 256)), specs_w((1, 256)),
            specs_w((256, 128)), specs_w((1, 128)),
            specs_w((1, 128)), specs_w((1, 128)),
        ],
        out_specs=pl.BlockSpec((_ET, 128), lambda i: (i, 0)),
        out_shape=jax.ShapeDtypeStruct((_E, 128), _f32),
    )(dists, pe, w1[:400], w1[400:], b1[None], pp['l2']['w'], pp['l2']['b'][None],
      pp['l3']['w'], pp['l3']['b'][None], pp['ln']['g'][None], pp['ln']['b'][None])


def _expand_mat():
    rowi = lax.broadcasted_iota(jnp.int32, (_ET, _T), 0)
    coli = lax.broadcasted_iota(jnp.int32, (_ET, _T), 1)
    return (rowi // _K == coli).astype(_f32)


def _agg_mat():
    rowi = lax.broadcasted_iota(jnp.int32, (_T, _ET), 0)
    coli = lax.broadcasted_iota(jnp.int32, (_T, _ET), 1)
    return (coli // _K == rowi).astype(_f32)


def _msg_body(sg_ref, nd_ref, e_ref, rn_ref, wa_ref, wb_ref, wc_ref, wd_ref,
              b1_ref, w2_ref, b2_ref, w3_ref, b3_ref, out_ref):
    sdw = jnp.dot(_expand_mat(), _dot(nd_ref[...], wb_ref[...]),
                  preferred_element_type=_f32,
                  precision=lax.Precision.HIGHEST)
    m = jax.nn.relu(
        _dot(sg_ref[...], wa_ref[...]) + sdw
        + _dot(e_ref[...], wc_ref[...]) + _dot(rn_ref[...], wd_ref[...])
        + b1_ref[...])
    m = jax.nn.relu(_dot(m, w2_ref[...]) + b2_ref[...])
    m = _dot(m, w3_ref[...]) + b3_ref[...]
    out_ref[...] = jnp.dot(_agg_mat(), m, preferred_element_type=_f32,
                           precision=lax.Precision.HIGHEST) * (1.0 / _K)


def _message(sg, node, e, rn, p):
    w1, b1 = p['m1']['w'], p['m1']['b']
    specs_w = lambda shp: pl.BlockSpec(shp, lambda i: (0, 0))
    return pl.pallas_call(
        _msg_body,
        grid=(_GRID,),
        in_specs=[
            pl.BlockSpec((_ET, 128), lambda i: (i, 0)),
            pl.BlockSpec((_T, 128), lambda i: (i, 0)),
            pl.BlockSpec((_ET, 128), lambda i: (i, 0)),
            pl.BlockSpec((_ET, 32), lambda i: (i, 0)),
            specs_w((128, 128)), specs_w((128, 128)), specs_w((128, 128)),
            specs_w((32, 128)), specs_w((1, 128)),
            specs_w((128, 128)), specs_w((1, 128)),
            specs_w((128, 128)), specs_w((1, 128)),
        ],
        out_specs=pl.BlockSpec((_T, 128), lambda i: (i, 0)),
        out_shape=jax.ShapeDtypeStruct((_N, 128), _f32),
    )(sg, node, e, rn, w1[:128], w1[128:256], w1[256:384], w1[384:416],
      b1[None], p['m2']['w'], p['m2']['b'][None], p['m3']['w'], p['m3']['b'][None])


def _node_body(node_ref, upd_ref, msk_ref, g1_ref, bb1_ref, l1w_ref, l1b_ref,
               l2w_ref, l2b_ref, l3w_ref, l3b_ref, g2_ref, bb2_ref,
               iw_ref, ib_ref, node_out_ref, hb_ref):
    msk = msk_ref[...]
    n1 = _ln(node_ref[...] + upd_ref[...] * msk, g1_ref[...], bb1_ref[...])
    h = jax.nn.relu(_dot(n1, l1w_ref[...]) + l1b_ref[...])
    h = jax.nn.relu(_dot(h, l2w_ref[...]) + l2b_ref[...])
    h = _dot(h, l3w_ref[...]) + l3b_ref[...]
    n2 = _ln(n1 + h, g2_ref[...], bb2_ref[...]) * msk
    node_out_ref[...] = n2
    hb_ref[...] = _dot(n2, iw_ref[...]) + ib_ref[...]


def _node_update(node, upd, nmask, lp):
    nt, et = lp['nt'], lp['et']
    return pl.pallas_call(
        _node_body,
        out_shape=(jax.ShapeDtypeStruct((_N, 128), _f32),
                   jax.ShapeDtypeStruct((_N, 64), _f32)),
    )(node, upd, nmask, lp['ln']['g'][None], lp['ln']['b'][None],
      nt['l1']['w'], nt['l1']['b'][None], nt['l2']['w'], nt['l2']['b'][None],
      nt['l3']['w'], nt['l3']['b'][None], nt['ln']['g'][None], nt['ln']['b'][None],
      et['init']['w'], et['init']['b'][None])


def _etrans_body(hs_ref, hd_ref, e_ref, t1a_ref, t1b_ref, t1c_ref, b1_ref,
                 w2_ref, b2_ref, fa_ref, fb_ref, fc_ref, fw_ref, fbias_ref,
                 g_ref, bb_ref, out_ref):
    hs, hd, e = hs_ref[...], hd_ref[...], e_ref[...]
    P = _expand_mat()
    hdw1 = jnp.dot(P, _dot(hd, t1b_ref[...]), preferred_element_type=_f32,
                   precision=lax.Precision.HIGHEST)
    hdw2 = jnp.dot(P, _dot(hd, fb_ref[...]), preferred_element_type=_f32,
                   precision=lax.Precision.HIGHEST)
    t1 = jax.nn.relu(_dot(hs, t1a_ref[...]) + hdw1
                     + _dot(e, t1c_ref[...]) + b1_ref[...])
    t2 = jax.nn.relu(_dot(t1, w2_ref[...]) + b2_ref[...])
    x = (_dot(t2, fw_ref[...]) + _dot(hs, fa_ref[...]) + hdw2
         + _dot(e, fc_ref[...]) + fbias_ref[...])
    out_ref[...] = _ln(x, g_ref[...], bb_ref[...])


def _edge_transition(hs, hb, e, et):
    t1w, fw = et['t1']['w'], et['final']['w']
    specs_w = lambda shp: pl.BlockSpec(shp, lambda i: (0, 0))
    return pl.pallas_call(
        _etrans_body,
        grid=(_GRID,),
        in_specs=[
            pl.BlockSpec((_ET, 64), lambda i: (i, 0)),
            pl.BlockSpec((_T, 64), lambda i: (i, 0)),
            pl.BlockSpec((_ET, 128), lambda i: (i, 0)),
            specs_w((64, 256)), specs_w((64, 256)), specs_w((128, 256)),
            specs_w((1, 256)),
            specs_w((256, 256)), specs_w((1, 256)),
            specs_w((64, 128)), specs_w((64, 128)), specs_w((128, 128)),
            specs_w((256, 128)), specs_w((1, 128)),
            specs_w((1, 128)), specs_w((1, 128)),
        ],
        out_specs=pl.BlockSpec((_ET, 128), lambda i: (i, 0)),
        out_shape=jax.ShapeDtypeStruct((_E, 128), _f32),
    )(hs, hb, e, t1w[:64], t1w[64:128], t1w[128:], et['t1']['b'][None],
      et['t2']['w'], et['t2']['b'][None],
      fw[:64], fw[64:128], fw[128:], fw, et['final']['b'][None],
      et['ln']['g'][None], et['ln']['b'][None])


def _quat_to_rot(q):
    q = q / (jnp.linalg.norm(q, axis=-1, keepdims=True) + 1e-8)
    w, x, y, z = q[..., 0], q[..., 1], q[..., 2], q[..., 3]
    R = jnp.stack([1 - 2 * (y * y + z * z), 2 * (x * y - w * z), 2 * (x * z + w * y),
                   2 * (x * y + w * z), 1 - 2 * (x * x + z * z), 2 * (y * z - w * x),
                   2 * (x * z - w * y), 2 * (y * z + w * x), 1 - 2 * (x * x + y * y)], -1)
    return R.reshape(q.shape[:-1] + (3, 3))


def _rotx(tor):
    s_, c_ = tor[..., 0], tor[..., 1]
    z = jnp.zeros_like(s_)
    o = jnp.ones_like(s_)
    Rm = jnp.stack([o, z, z, z, c_, -s_, z, s_, c_], -1)
    return Rm.reshape(tor.shape[:-1] + (3, 3))


def kernel(X_ca, bb, x_mask, rigids_0, batch, latent, params):
    eps = 1e-8
    N = _N
    n_, ca, c_ = bb[:, 0], bb[:, 1], bb[:, 2]
    b_ = ca - n_
    cc_ = c_ - ca
    a_ = jnp.cross(b_, cc_)
    vcb = -0.58273431 * a_ + 0.56802827 * b_ - 0.54067466 * cc_ + ca
    bb5 = jnp.concatenate([bb, vcb[:, None, :]], axis=1)
    R = _quat_to_rot(rigids_0[:, :4])
    t = rigids_0[:, 4:7]

    Xm = jnp.where(x_mask[:, None], 1e6, X_ca)
    d2 = jnp.sum((Xm[:, None, :] - Xm[None, :, :]) ** 2, -1)
    valid = (batch[:, None] == batch[None, :]) & (~jnp.eye(N, dtype=bool))
    d2 = jnp.where(valid, d2, jnp.inf)
    _, nbr = jax.lax.top_k(-d2, _K)
    src = nbr.reshape(-1).astype(jnp.int32)
    dst_nodes = jnp.arange(N, dtype=jnp.int32)

    bb5f = bb5.reshape(N, 15)
    bb5_pad = jnp.concatenate([bb5f, jnp.zeros((N, 113), _f32)], -1)
    ebd = _sc_gather(bb5_pad, src, 128)[:, :15].reshape(_E, 5, 3)
    ebs = jnp.repeat(bb5, _K, axis=0)
    diff = ebs[:, :, None, :] - ebd[:, None, :, :] + eps
    dists = jnp.sqrt(jnp.sum(diff * diff, -1)).reshape(_E, 25)
    d_ = (src - jnp.repeat(dst_nodes, _K)).astype(_f32)
    freq = jnp.exp(jnp.arange(0, 16, 2, dtype=_f32) * (-np.log(10000.0) / 16))
    ang = d_[:, None] * freq[None, :]
    pe = jnp.concatenate([jnp.cos(ang), jnp.sin(ang)], -1)
    e = _edge_embed(dists, pe, params['embed_edge'])

    node = latent
    nmask = (~x_mask).astype(_f32)[:, None]
    n_layers = len(params['layers'])
    for li, lp in enumerate(params['layers']):
        ip = lp['ipmp']
        pts = (node @ ip['q_pts']['w'] + ip['q_pts']['b']).reshape(N, 8, 3)
        pts_g = jnp.einsum('nij,npj->npi', R, pts) + t[:, None, :]
        table = jnp.concatenate(
            [node, pts_g.reshape(N, 24), jnp.zeros((N, 104), _f32)], -1)
        g = _sc_gather(table, src, 256)
        sg = g[:, :128]
        ptsg_src = g[:, 128:152].reshape(_E, 8, 3)
        R_e = jnp.repeat(R, _K, axis=0)
        t_e = jnp.repeat(t, _K, axis=0)
        rel = jnp.einsum('eji,epj->epi', R_e, ptsg_src - t_e[:, None, :])
        nrm = jnp.sqrt(jnp.sum((rel + eps) ** 2, -1))
        rn = jnp.concatenate([rel.reshape(_E, 24), nrm], -1)
        upd = _message(sg, node, e, rn, ip)
        node, hb = _node_update(node, upd, nmask, lp)
        if li < n_layers - 1:
            hs = _sc_gather(
                jnp.concatenate([hb, jnp.zeros((_N, 64), _f32)], -1), src, 128
            )[:, :64]
            e = _edge_transition(hs, hb, e, lp['et'])

    unnorm = (node @ params['torsion_pred']['w'] + params['torsion_pred']['b']).reshape(-1, 81, 2)
    tors = unnorm / jnp.linalg.norm(unnorm + 1e-8, axis=-1, keepdims=True)
    psi = tors[:, :1, :]
    chis = tors[:, 1:, :].reshape(-1, 20, 4, 2)
    ident = jnp.broadcast_to(jnp.eye(3), (N, 20, 1, 3, 3))
    Rpsi = jnp.broadcast_to(_rotx(psi)[:, None], (N, 20, 1, 3, 3))
    Rchi = _rotx(chis)
    rots = jnp.concatenate([ident, Rpsi, Rchi], axis=2)
    ratoms = rots[:, jnp.arange(20)[:, None], _GROUP_IDX_]
    local = jnp.einsum('nafij,afj->nafi', ratoms, _LIT_POS_)
    all14 = jnp.einsum('nij,nafj->nafi', R, local) + t[:, None, None, :]
    atom91 = jnp.zeros((N, 91, 3), all14.dtype)
    atom91 = atom91.at[:, :4, :].set(all14[:, 0, :4, :])
    for i in range(20):
        s0, e0 = int(_STARTS_[i]), int(_ENDS_[i])
        if e0 > s0:
            atom91 = atom91.at[:, s0:e0, :].set(all14[:, i, 4:4 + (e0 - s0), :])
    atom91 = atom91 - t[:, None, :]
    seq_logits = node @ params['seq_head']['w'] + params['seq_head']['b']
    return atom91, seq_logits


# speedup vs baseline: 1.0283x; 1.0283x over previous
import functools

import jax
import jax.numpy as jnp
import numpy as np
from jax import lax
from jax.experimental import pallas as pl
from jax.experimental.pallas import tpu as pltpu
from jax.experimental.pallas import tpu_sc as plsc

_N = 4096
_K = 30
_E = _N * _K
_T = 128
_ET = _T * _K
_GRID = _N // _T
_NUM_RBF = 16

_SC_COUNTS_ = np.array([1, 7, 4, 4, 2, 5, 5, 0, 6, 4, 4, 5, 4, 7, 3, 2, 3, 10, 8, 3])
_cum_ = np.concatenate([[0], np.cumsum(_SC_COUNTS_)])
_STARTS_ = 4 + _cum_[:-1]
_ENDS_ = 4 + _cum_[1:]
_crng_ = np.random.RandomState(7)
_LIT_POS_ = jnp.asarray(_crng_.randn(20, 14, 3).astype(np.float32))
_GROUP_IDX_ = jnp.asarray(_crng_.randint(0, 6, size=(20, 14)).astype(np.int32))

_f32 = jnp.float32


def _dot(a, b):
    return jnp.dot(a, b, preferred_element_type=_f32)


def _ln(x, g, b):
    mu = jnp.mean(x, -1, keepdims=True)
    var = jnp.mean((x - mu) ** 2, -1, keepdims=True)
    return (x - mu) / jnp.sqrt(var + 1e-5) * g + b


_NW = 32
_BPW = _E // _NW
_CH = 128
_NCH = _BPW // _CH


@functools.partial(jax.jit, static_argnums=(2,))
def _sc_gather(table, idx, d):
    mesh = plsc.VectorSubcoreMesh(core_axis_name="c", subcore_axis_name="s")

    @functools.partial(
        pl.kernel,
        mesh=mesh,
        out_type=jax.ShapeDtypeStruct((_E, d), _f32),
        scratch_types=[
            pltpu.VMEM((_BPW,), jnp.int32),
            pltpu.VMEM((_CH, d), _f32),
            pltpu.SemaphoreType.DMA,
        ],
    )
    def k(table_hbm, idx_hbm, out_hbm, idx_v, rows_v, sem):
        wid = lax.axis_index("s") * 2 + lax.axis_index("c")
        base = wid * _BPW
        pltpu.sync_copy(idx_hbm.at[pl.ds(base, _BPW)], idx_v)

        def body(j, carry):
            pltpu.async_copy(
                table_hbm.at[idx_v.at[pl.ds(j * _CH, _CH)]], rows_v, sem
            ).wait()
            pltpu.sync_copy(rows_v, out_hbm.at[pl.ds(base + j * _CH, _CH)])
            return carry

        lax.fori_loop(0, _NCH, body, 0)

    return k(table, idx)


def _embed_body(dists_ref, pe_ref, w1a_ref, w1b_ref, b1_ref, w2_ref, b2_ref,
                w3_ref, b3_ref, g_ref, bb_ref, out_ref):
    dists = dists_ref[...]
    col = lax.broadcasted_iota(jnp.int32, (25, 400), 1)
    row = lax.broadcasted_iota(jnp.int32, (25, 400), 0)
    S = (col // _NUM_RBF == row).astype(_f32)
    dexp = _dot(dists, S)
    cc = lax.broadcasted_iota(jnp.int32, (1, 400), 1)
    mu = 2.0 + (cc % _NUM_RBF).astype(_f32) * (20.0 / (_NUM_RBF - 1))
    rbf = jnp.exp(-(((dexp - mu) / 1.25) ** 2))
    h = jax.nn.relu(_dot(rbf, w1a_ref[...]) + _dot(pe_ref[...], w1b_ref[...])
                    + b1_ref[...])
    h = jax.nn.relu(_dot(h, w2_ref[...]) + b2_ref[...])
    h = _dot(h, w3_ref[...]) + b3_ref[...]
    out_ref[...] = _ln(h, g_ref[...], bb_ref[...])


def _edge_embed(dists, pe, pp):
    w1, b1 = pp['l1']['w'], pp['l1']['b']
    specs_w = lambda shp: pl.BlockSpec(shp, lambda i: (0, 0))
    return pl.pallas_call(
        _embed_body,
        grid=(_GRID,),
        in_specs=[
            pl.BlockSpec((_ET, 25), lambda i: (i, 0)),
            pl.BlockSpec((_ET, 16), lambda i: (i, 0)),
            specs_w((400, 256)), specs_w((16, 256)), specs_w((1, 256)),
            specs_w((256, 256)), specs_w((1, 256)),
            specs_w((256, 128)), specs_w((1, 128)),
            specs_w((1, 128)), specs_w((1, 128)),
        ],
        out_specs=pl.BlockSpec((_ET, 128), lambda i: (i, 0)),
        out_shape=jax.ShapeDtypeStruct((_E, 128), _f32),
    )(dists, pe, w1[:400], w1[400:], b1[None], pp['l2']['w'], pp['l2']['b'][None],
      pp['l3']['w'], pp['l3']['b'][None], pp['ln']['g'][None], pp['ln']['b'][None])


def _msg_body(sg_ref, sd_ref, e_ref, rn_ref, wa_ref, wb_ref, wc_ref, wd_ref,
              b1_ref, w2_ref, b2_ref, w3_ref, b3_ref, agg_ref, out_ref):
    m = jax.nn.relu(
        _dot(sg_ref[...], wa_ref[...]) + _dot(sd_ref[...], wb_ref[...])
        + _dot(e_ref[...], wc_ref[...]) + _dot(rn_ref[...], wd_ref[...])
        + b1_ref[...])
    m = jax.nn.relu(_dot(m, w2_ref[...]) + b2_ref[...])
    m = _dot(m, w3_ref[...]) + b3_ref[...]
    out_ref[...] = _dot(agg_ref[...], m) * (1.0 / _K)


def _message(sg, sd_rep, e, rn, p, aggmat):
    w1, b1 = p['m1']['w'], p['m1']['b']
    specs_w = lambda shp: pl.BlockSpec(shp, lambda i: (0, 0))
    return pl.pallas_call(
        _msg_body,
        grid=(_GRID,),
        in_specs=[
            pl.BlockSpec((_ET, 128), lambda i: (i, 0)),
            pl.BlockSpec((_ET, 128), lambda i: (i, 0)),
            pl.BlockSpec((_ET, 128), lambda i: (i, 0)),
            pl.BlockSpec((_ET, 32), lambda i: (i, 0)),
            specs_w((128, 128)), specs_w((128, 128)), specs_w((128, 128)),
            specs_w((32, 128)), specs_w((1, 128)),
            specs_w((128, 128)), specs_w((1, 128)),
            specs_w((128, 128)), specs_w((1, 128)),
            specs_w((_T, _ET)),
        ],
        out_specs=pl.BlockSpec((_T, 128), lambda i: (i, 0)),
        out_shape=jax.ShapeDtypeStruct((_N, 128), _f32),
    )(sg, sd_rep, e, rn, w1[:128], w1[128:256], w1[256:384], w1[384:416],
      b1[None], p['m2']['w'], p['m2']['b'][None], p['m3']['w'], p['m3']['b'][None],
      aggmat)


def _node_body(node_ref, upd_ref, msk_ref, g1_ref, bb1_ref, l1w_ref, l1b_ref,
               l2w_ref, l2b_ref, l3w_ref, l3b_ref, g2_ref, bb2_ref,
               iw_ref, ib_ref, node_out_ref, hb_ref):
    msk = msk_ref[...]
    n1 = _ln(node_ref[...] + upd_ref[...] * msk, g1_ref[...], bb1_ref[...])
    h = jax.nn.relu(_dot(n1, l1w_ref[...]) + l1b_ref[...])
    h = jax.nn.relu(_dot(h, l2w_ref[...]) + l2b_ref[...])
    h = _dot(h, l3w_ref[...]) + l3b_ref[...]
    n2 = _ln(n1 + h, g2_ref[...], bb2_ref[...]) * msk
    node_out_ref[...] = n2
    hb_ref[...] = _dot(n2, iw_ref[...]) + ib_ref[...]


def _node_update(node, upd, nmask, lp):
    nt, et = lp['nt'], lp['et']
    return pl.pallas_call(
        _node_body,
        out_shape=(jax.ShapeDtypeStruct((_N, 128), _f32),
                   jax.ShapeDtypeStruct((_N, 64), _f32)),
    )(node, upd, nmask, lp['ln']['g'][None], lp['ln']['b'][None],
      nt['l1']['w'], nt['l1']['b'][None], nt['l2']['w'], nt['l2']['b'][None],
      nt['l3']['w'], nt['l3']['b'][None], nt['ln']['g'][None], nt['ln']['b'][None],
      et['init']['w'], et['init']['b'][None])


def _etrans_body(hs_ref, hd_ref, e_ref, t1a_ref, t1b_ref, t1c_ref, b1_ref,
                 w2_ref, b2_ref, fa_ref, fb_ref, fc_ref, fw_ref, fbias_ref,
                 g_ref, bb_ref, out_ref):
    hs, hd, e = hs_ref[...], hd_ref[...], e_ref[...]
    t1 = jax.nn.relu(_dot(hs, t1a_ref[...]) + _dot(hd, t1b_ref[...])
                     + _dot(e, t1c_ref[...]) + b1_ref[...])
    t2 = jax.nn.relu(_dot(t1, w2_ref[...]) + b2_ref[...])
    x = (_dot(t2, fw_ref[...]) + _dot(hs, fa_ref[...]) + _dot(hd, fb_ref[...])
         + _dot(e, fc_ref[...]) + fbias_ref[...])
    out_ref[...] = _ln(x, g_ref[...], bb_ref[...])


def _edge_transition(hs, hd_rep, e, et):
    t1w, fw = et['t1']['w'], et['final']['w']
    specs_w = lambda shp: pl.BlockSpec(shp, lambda i: (0, 0))
    return pl.pallas_call(
        _etrans_body,
        grid=(_GRID,),
        in_specs=[
            pl.BlockSpec((_ET, 64), lambda i: (i, 0)),
            pl.BlockSpec((_ET, 64), lambda i: (i, 0)),
            pl.BlockSpec((_ET, 128), lambda i: (i, 0)),
            specs_w((64, 256)), specs_w((64, 256)), specs_w((128, 256)),
            specs_w((1, 256)),
            specs_w((256, 256)), specs_w((1, 256)),
            specs_w((64, 128)), specs_w((64, 128)), specs_w((128, 128)),
            specs_w((256, 128)), specs_w((1, 128)),
            specs_w((1, 128)), specs_w((1, 128)),
        ],
        out_specs=pl.BlockSpec((_ET, 128), lambda i: (i, 0)),
        out_shape=jax.ShapeDtypeStruct((_E, 128), _f32),
    )(hs, hd_rep, e, t1w[:64], t1w[64:128], t1w[128:], et['t1']['b'][None],
      et['t2']['w'], et['t2']['b'][None],
      fw[:64], fw[64:128], fw[128:], fw, et['final']['b'][None],
      et['ln']['g'][None], et['ln']['b'][None])


def _quat_to_rot(q):
    q = q / (jnp.linalg.norm(q, axis=-1, keepdims=True) + 1e-8)
    w, x, y, z = q[..., 0], q[..., 1], q[..., 2], q[..., 3]
    R = jnp.stack([1 - 2 * (y * y + z * z), 2 * (x * y - w * z), 2 * (x * z + w * y),
                   2 * (x * y + w * z), 1 - 2 * (x * x + z * z), 2 * (y * z - w * x),
                   2 * (x * z - w * y), 2 * (y * z + w * x), 1 - 2 * (x * x + y * y)], -1)
    return R.reshape(q.shape[:-1] + (3, 3))


def _rotx(tor):
    s_, c_ = tor[..., 0], tor[..., 1]
    z = jnp.zeros_like(s_)
    o = jnp.ones_like(s_)
    Rm = jnp.stack([o, z, z, z, c_, -s_, z, s_, c_], -1)
    return Rm.reshape(tor.shape[:-1] + (3, 3))


def kernel(X_ca, bb, x_mask, rigids_0, batch, latent, params):
    eps = 1e-8
    N = _N
    n_, ca, c_ = bb[:, 0], bb[:, 1], bb[:, 2]
    b_ = ca - n_
    cc_ = c_ - ca
    a_ = jnp.cross(b_, cc_)
    vcb = -0.58273431 * a_ + 0.56802827 * b_ - 0.54067466 * cc_ + ca
    bb5 = jnp.concatenate([bb, vcb[:, None, :]], axis=1)
    R = _quat_to_rot(rigids_0[:, :4])
    t = rigids_0[:, 4:7]

    Xm = jnp.where(x_mask[:, None], 1e6, X_ca)
    d2 = jnp.sum((Xm[:, None, :] - Xm[None, :, :]) ** 2, -1)
    valid = (batch[:, None] == batch[None, :]) & (~jnp.eye(N, dtype=bool))
    d2 = jnp.where(valid, d2, jnp.inf)
    _, nbr = jax.lax.top_k(-d2, _K)
    src = nbr.reshape(-1).astype(jnp.int32)
    dst_nodes = jnp.arange(N, dtype=jnp.int32)

    bb5f = bb5.reshape(N, 15)
    bb5_pad = jnp.concatenate([bb5f, jnp.zeros((N, 113), _f32)], -1)
    ebd = _sc_gather(bb5_pad, src, 128)[:, :15].reshape(_E, 5, 3)
    ebs = jnp.repeat(bb5, _K, axis=0)
    diff = ebs[:, :, None, :] - ebd[:, None, :, :] + eps
    dists = jnp.sqrt(jnp.sum(diff * diff, -1)).reshape(_E, 25)
    d_ = (src - jnp.repeat(dst_nodes, _K)).astype(_f32)
    freq = jnp.exp(jnp.arange(0, 16, 2, dtype=_f32) * (-np.log(10000.0) / 16))
    ang = d_[:, None] * freq[None, :]
    pe = jnp.concatenate([jnp.cos(ang), jnp.sin(ang)], -1)
    e = _edge_embed(dists, pe, params['embed_edge'])

    rowi = lax.broadcasted_iota(jnp.int32, (_T, _ET), 0)
    coli = lax.broadcasted_iota(jnp.int32, (_T, _ET), 1)
    aggmat = (coli // _K == rowi).astype(_f32)

    node = latent
    nmask = (~x_mask).astype(_f32)[:, None]
    n_layers = len(params['layers'])
    for li, lp in enumerate(params['layers']):
        ip = lp['ipmp']
        pts = (node @ ip['q_pts']['w'] + ip['q_pts']['b']).reshape(N, 8, 3)
        pts_g = jnp.einsum('nij,npj->npi', R, pts) + t[:, None, :]
        table = jnp.concatenate(
            [node, pts_g.reshape(N, 24), jnp.zeros((N, 104), _f32)], -1)
        g = _sc_gather(table, src, 256)
        sg = g[:, :128]
        ptsg_src = g[:, 128:152].reshape(_E, 8, 3)
        R_e = jnp.repeat(R, _K, axis=0)
        t_e = jnp.repeat(t, _K, axis=0)
        rel = jnp.einsum('eji,epj->epi', R_e, ptsg_src - t_e[:, None, :])
        nrm = jnp.sqrt(jnp.sum((rel + eps) ** 2, -1))
        rn = jnp.concatenate([rel.reshape(_E, 24), nrm], -1)
        sd_rep = jnp.repeat(node, _K, axis=0)
        upd = _message(sg, sd_rep, e, rn, ip, aggmat)
        node, hb = _node_update(node, upd, nmask, lp)
        if li < n_layers - 1:
            hs = _sc_gather(
                jnp.concatenate([hb, jnp.zeros((_N, 64), _f32)], -1), src, 128
            )[:, :64]
            hd_rep = jnp.repeat(hb, _K, axis=0)
            e = _edge_transition(hs, hd_rep, e, lp['et'])

    unnorm = (node @ params['torsion_pred']['w'] + params['torsion_pred']['b']).reshape(-1, 81, 2)
    tors = unnorm / jnp.linalg.norm(unnorm + 1e-8, axis=-1, keepdims=True)
    psi = tors[:, :1, :]
    chis = tors[:, 1:, :].reshape(-1, 20, 4, 2)
    ident = jnp.broadcast_to(jnp.eye(3), (N, 20, 1, 3, 3))
    Rpsi = jnp.broadcast_to(_rotx(psi)[:, None], (N, 20, 1, 3, 3))
    Rchi = _rotx(chis)
    rots = jnp.concatenate([ident, Rpsi, Rchi], axis=2)
    ratoms = rots[:, jnp.arange(20)[:, None], _GROUP_IDX_]
    local = jnp.einsum('nafij,afj->nafi', ratoms, _LIT_POS_)
    all14 = jnp.einsum('nij,nafj->nafi', R, local) + t[:, None, None, :]
    atom91 = jnp.zeros((N, 91, 3), all14.dtype)
    atom91 = atom91.at[:, :4, :].set(all14[:, 0, :4, :])
    for i in range(20):
        s0, e0 = int(_STARTS_[i]), int(_ENDS_[i])
        if e0 > s0:
            atom91 = atom91.at[:, s0:e0, :].set(all14[:, i, 4:4 + (e0 - s0), :])
    atom91 = atom91 - t[:, None, :]
    seq_logits = node @ params['seq_head']['w'] + params['seq_head']['b']
    return atom91, seq_logits
